# Initial kernel scaffold; baseline (speedup 1.0000x reference)
import jax
import jax.numpy as jnp
from jax.experimental import pallas as pl


def kernel(x, edge_index, batch, Wrel1, Wroot1, b1, Wrel2, Wroot2, b2, Wrel3, Wroot3, b3, Wrel4, Wroot4, b4, lin1_W, lin1_b, lin2_W, lin2_b):
    raise NotImplementedError("write your pallas kernel here")


# speedup vs baseline: 5.8988x; 5.8988x over previous
import functools

import jax
import jax.numpy as jnp
from jax import lax
from jax.experimental import pallas as pl
from jax.experimental.pallas import tpu as pltpu
from jax.experimental.pallas import tpu_sc as plsc

N = 10000
E = 320000
G = 128
NSC = 16
K = 80
NPAD = 10240
NPS = NPAD // NSC
R = 1000
NCH_C = E // NSC // K
NCH_E = E // 2 // NSC // K
NB = 25

_MESH = plsc.VectorSubcoreMesh(core_axis_name="c", subcore_axis_name="s")


def _edge_loop(c, s, src5, dst5, hh, acc, srcv, dstv, rows, gsem, ssem, nblk):
    def blk_body(blk, carry):
        pltpu.sync_copy(src5.at[c, s, blk], srcv)
        pltpu.sync_copy(dst5.at[c, s, blk], dstv)

        def body(j, carry2):
            pltpu.async_copy(hh.at[srcv.at[j]], rows, gsem).wait()
            pltpu.async_copy(rows, acc.at[dstv.at[j]], ssem, add=True).wait()
            return carry2

        return lax.fori_loop(0, NB, body, carry)

    lax.fori_loop(0, nblk, blk_body, 0)


def _make_sc_agg(nch):
    nblk = nch // NB

    @functools.partial(
        pl.kernel,
        out_type=jax.ShapeDtypeStruct((2, NPAD, 128), jnp.float32),
        mesh=_MESH,
        scratch_types=[
            pltpu.VMEM((NB, K), jnp.int32),
            pltpu.VMEM((NB, K), jnp.int32),
            pltpu.VMEM((K, 128), jnp.float32),
            pltpu.VMEM_SHARED((NPAD, 128), jnp.float32),
            pltpu.SemaphoreType.DMA,
            pltpu.SemaphoreType.DMA,
        ],
    )
    def agg(table, src5, dst5, zrows, out, srcv, dstv, rows, acc, gsem, ssem):
        c = lax.axis_index("c")
        s = lax.axis_index("s")
        pltpu.sync_copy(zrows, acc.at[pl.ds(s * NPS, NPS)])
        plsc.subcore_barrier()
        _edge_loop(c, s, src5, dst5, table, acc, srcv, dstv, rows,
                   gsem, ssem, nblk)
        plsc.subcore_barrier()
        pltpu.sync_copy(acc.at[pl.ds(s * NPS, NPS)],
                        out.at[c, pl.ds(s * NPS, NPS)])

    return agg


_sc_agg_edges = _make_sc_agg(NCH_E)
_sc_agg_cols = _make_sc_agg(NCH_C)


def _tc_layer(agg2, h, wrel, wroot, b, relu, agg_sum, out_halves):
    fin, fout = wrel.shape
    h_halves = h.ndim == 3
    nblk = N // R

    def body(agg_ref, h_ref, wrel_ref, wroot_ref, b_ref, out_ref):
        if agg_sum:
            agg = agg_ref[0] + agg_ref[1]
        else:
            agg = jnp.concatenate([agg_ref[0], agg_ref[1]], axis=-1)
        if h_halves:
            hb = jnp.concatenate([h_ref[0], h_ref[1]], axis=-1)
        else:
            hb = h_ref[...]
        y = (jnp.dot(agg, wrel_ref[...], preferred_element_type=jnp.float32)
             + jnp.dot(hb, wroot_ref[...], preferred_element_type=jnp.float32)
             + b_ref[...])
        if relu:
            y = jnp.maximum(y, 0.0)
        if out_halves:
            out_ref[0] = y[:, :fout // 2]
            out_ref[1] = y[:, fout // 2:]
        else:
            out_ref[...] = y

    h_spec = (pl.BlockSpec((2, R, 128), lambda i: (0, i, 0)) if h_halves
              else pl.BlockSpec((R, fin), lambda i: (i, 0)))
    out_spec = (pl.BlockSpec((2, R, fout // 2), lambda i: (0, i, 0))
                if out_halves else pl.BlockSpec((R, fout), lambda i: (i, 0)))
    out_shape = (jax.ShapeDtypeStruct((2, N, fout // 2), jnp.float32)
                 if out_halves else jax.ShapeDtypeStruct((N, fout), jnp.float32))
    return pl.pallas_call(
        body,
        grid=(nblk,),
        in_specs=[
            pl.BlockSpec((2, R, 128), lambda i: (0, i, 0)),
            h_spec,
            pl.BlockSpec((fin, fout), lambda i: (0, 0)),
            pl.BlockSpec((fin, fout), lambda i: (0, 0)),
            pl.BlockSpec((1, fout), lambda i: (0, 0)),
        ],
        out_specs=out_spec,
        out_shape=out_shape,
    )(agg2, h, wrel, wroot, b.reshape(1, fout))


def _tc_pool_mlp(h2, batch3, w1, b1, w2p, b2p):
    nblk = N // R

    def body(h_ref, bat_ref, w1_ref, b1_ref, w2_ref, b2_ref, out_ref,
             pooled_ref, cnt_ref):
        i = pl.program_id(0)

        @pl.when(i == 0)
        def _():
            pooled_ref[...] = jnp.zeros_like(pooled_ref)
            cnt_ref[...] = jnp.zeros_like(cnt_ref)

        h = jnp.concatenate([h_ref[0], h_ref[1]], axis=-1)
        bvec = bat_ref[0, 0, :]
        onehot = (bvec[:, None]
                  == lax.broadcasted_iota(jnp.int32, (R, G), 1)
                  ).astype(jnp.float32)
        pooled_ref[...] += lax.dot_general(
            onehot, h, (((0,), (0,)), ((), ())),
            preferred_element_type=jnp.float32)
        cnt_ref[0, :] += jnp.sum(onehot, axis=0)

        @pl.when(i == nblk - 1)
        def _():
            cnt = jnp.maximum(cnt_ref[0, :], 1.0)
            pooled = pooled_ref[...] / cnt[:, None]
            t = (jnp.dot(pooled, w1_ref[...],
                         preferred_element_type=jnp.float32) + b1_ref[...])
            out_ref[...] = (jnp.dot(t, w2_ref[...],
                                    preferred_element_type=jnp.float32)
                            + b2_ref[...])

    return pl.pallas_call(
        body,
        grid=(nblk,),
        in_specs=[
            pl.BlockSpec((2, R, 128), lambda i: (0, i, 0)),
            pl.BlockSpec((1, 1, R), lambda i: (i, 0, 0)),
            pl.BlockSpec((256, 128), lambda i: (0, 0)),
            pl.BlockSpec((1, 128), lambda i: (0, 0)),
            pl.BlockSpec((128, 128), lambda i: (0, 0)),
            pl.BlockSpec((1, 128), lambda i: (0, 0)),
        ],
        out_specs=pl.BlockSpec((G, 128), lambda i: (0, 0)),
        out_shape=jax.ShapeDtypeStruct((G, 128), jnp.float32),
        scratch_shapes=[
            pltpu.VMEM((G, 256), jnp.float32),
            pltpu.VMEM((8, 128), jnp.float32),
        ],
    )(h2, batch3, w1, b1.reshape(1, 128), w2p, b2p)


def kernel(x, edge_index, batch, Wrel1, Wroot1, b1, Wrel2, Wroot2, b2,
           Wrel3, Wroot3, b3, Wrel4, Wroot4, b4, lin1_W, lin1_b,
           lin2_W, lin2_b):
    src, dst = edge_index[0], edge_index[1]
    src4 = src.reshape(2, NSC, NCH_E // NB, NB, K)
    dst4 = dst.reshape(2, NSC, NCH_E // NB, NB, K)
    src4c = jnp.stack([src, src + N]).reshape(2, NSC, NCH_C // NB, NB, K)
    dst4c = jnp.stack([dst, dst]).reshape(2, NSC, NCH_C // NB, NB, K)
    z128 = jnp.zeros((NPS, 128), jnp.float32)
    batch3 = batch.reshape(N // R, 1, R)

    agg = _sc_agg_edges(x, src4, dst4, z128)
    h = _tc_layer(agg, x, Wrel1, Wroot1, b1,
                  relu=True, agg_sum=True, out_halves=False)
    agg = _sc_agg_edges(h, src4, dst4, z128)
    h = _tc_layer(agg, h, Wrel2, Wroot2, b2,
                  relu=True, agg_sum=True, out_halves=True)
    agg = _sc_agg_cols(h.reshape(2 * N, 128), src4c, dst4c, z128)
    h = _tc_layer(agg, h, Wrel3, Wroot3, b3,
                  relu=True, agg_sum=False, out_halves=True)
    agg = _sc_agg_cols(h.reshape(2 * N, 128), src4c, dst4c, z128)
    h = _tc_layer(agg, h, Wrel4, Wroot4, b4,
                  relu=False, agg_sum=False, out_halves=True)

    w2p = jnp.zeros((128, 128), jnp.float32).at[:, :10].set(lin2_W)
    b2p = jnp.zeros((1, 128), jnp.float32).at[0, :10].set(lin2_b)
    out = _tc_pool_mlp(h, batch3, lin1_W, lin1_b, w2p, b2p)
    return out[:, :10]


# speedup vs baseline: 9.3719x; 1.5888x over previous
import functools

import jax
import jax.numpy as jnp
from jax import lax
from jax.experimental import pallas as pl
from jax.experimental.pallas import tpu as pltpu
from jax.experimental.pallas import tpu_sc as plsc

N = 10000
E = 320000
G = 128
NSC = 16
K = 80
NPAD = 10240
NPS = NPAD // NSC
R = 1000
NCH_C = E // NSC // K
NCH_E = E // 2 // NSC // K
NB = 25

_MESH = plsc.VectorSubcoreMesh(core_axis_name="c", subcore_axis_name="s")


def _edge_loop(c, s, src5, dst5, hh, acc, srcv, dstv, rows_a, rows_b,
               gsem_a, gsem_b, ssem, nblk):
    def wait_gather(j, buf, sem):
        pltpu.make_async_copy(hh.at[srcv.at[j]], buf, sem).wait()

    def scatter(j, buf):
        pltpu.async_copy(buf, acc.at[dstv.at[j]], ssem, add=True).wait()

    def blk_body(blk, carry):
        pltpu.sync_copy(src5.at[c, s, blk], srcv)
        pltpu.sync_copy(dst5.at[c, s, blk], dstv)
        pltpu.async_copy(hh.at[srcv.at[0]], rows_a, gsem_a)

        def pair(p, carry2):
            j0 = 2 * p
            pltpu.async_copy(hh.at[srcv.at[j0 + 1]], rows_b, gsem_b)
            wait_gather(j0, rows_a, gsem_a)
            scatter(j0, rows_a)
            pltpu.async_copy(hh.at[srcv.at[j0 + 2]], rows_a, gsem_a)
            wait_gather(j0 + 1, rows_b, gsem_b)
            scatter(j0 + 1, rows_b)
            return carry2

        lax.fori_loop(0, (NB - 1) // 2, pair, carry)
        wait_gather(NB - 1, rows_a, gsem_a)
        scatter(NB - 1, rows_a)
        return carry

    lax.fori_loop(0, nblk, blk_body, 0)


def _make_sc_agg(nch):
    nblk = nch // NB

    @functools.partial(
        pl.kernel,
        out_type=jax.ShapeDtypeStruct((2, NPAD, 128), jnp.float32),
        mesh=_MESH,
        scratch_types=[
            pltpu.VMEM((NB, K), jnp.int32),
            pltpu.VMEM((NB, K), jnp.int32),
            pltpu.VMEM((K, 128), jnp.float32),
            pltpu.VMEM((K, 128), jnp.float32),
            pltpu.VMEM_SHARED((NPAD, 128), jnp.float32),
            pltpu.SemaphoreType.DMA,
            pltpu.SemaphoreType.DMA,
            pltpu.SemaphoreType.DMA,
        ],
    )
    def agg(table, src5, dst5, zrows, out, srcv, dstv, rows_a, rows_b, acc,
            gsem_a, gsem_b, ssem):
        c = lax.axis_index("c")
        s = lax.axis_index("s")
        pltpu.sync_copy(zrows, acc.at[pl.ds(s * NPS, NPS)])
        plsc.subcore_barrier()
        _edge_loop(c, s, src5, dst5, table, acc, srcv, dstv, rows_a, rows_b,
                   gsem_a, gsem_b, ssem, nblk)
        plsc.subcore_barrier()
        pltpu.sync_copy(acc.at[pl.ds(s * NPS, NPS)],
                        out.at[c, pl.ds(s * NPS, NPS)])

    return agg


_sc_agg_edges = _make_sc_agg(NCH_E)
_sc_agg_cols = _make_sc_agg(NCH_C)


def _tc_layer(agg2, h, wrel, wroot, b, relu, agg_sum, out_halves):
    fin, fout = wrel.shape
    h_halves = h.ndim == 3
    nblk = N // R

    def body(agg_ref, h_ref, wrel_ref, wroot_ref, b_ref, out_ref):
        if agg_sum:
            agg = agg_ref[0] + agg_ref[1]
        else:
            agg = jnp.concatenate([agg_ref[0], agg_ref[1]], axis=-1)
        if h_halves:
            hb = jnp.concatenate([h_ref[0], h_ref[1]], axis=-1)
        else:
            hb = h_ref[...]
        y = (jnp.dot(agg, wrel_ref[...], preferred_element_type=jnp.float32)
             + jnp.dot(hb, wroot_ref[...], preferred_element_type=jnp.float32)
             + b_ref[...])
        if relu:
            y = jnp.maximum(y, 0.0)
        if out_halves:
            out_ref[0] = y[:, :fout // 2]
            out_ref[1] = y[:, fout // 2:]
        else:
            out_ref[...] = y

    h_spec = (pl.BlockSpec((2, R, 128), lambda i: (0, i, 0)) if h_halves
              else pl.BlockSpec((R, fin), lambda i: (i, 0)))
    out_spec = (pl.BlockSpec((2, R, fout // 2), lambda i: (0, i, 0))
                if out_halves else pl.BlockSpec((R, fout), lambda i: (i, 0)))
    out_shape = (jax.ShapeDtypeStruct((2, N, fout // 2), jnp.float32)
                 if out_halves else jax.ShapeDtypeStruct((N, fout), jnp.float32))
    return pl.pallas_call(
        body,
        grid=(nblk,),
        in_specs=[
            pl.BlockSpec((2, R, 128), lambda i: (0, i, 0)),
            h_spec,
            pl.BlockSpec((fin, fout), lambda i: (0, 0)),
            pl.BlockSpec((fin, fout), lambda i: (0, 0)),
            pl.BlockSpec((1, fout), lambda i: (0, 0)),
        ],
        out_specs=out_spec,
        out_shape=out_shape,
    )(agg2, h, wrel, wroot, b.reshape(1, fout))


def _tc_pool_mlp(h2, batch3, w1, b1, w2p, b2p):
    nblk = N // R

    def body(h_ref, bat_ref, w1_ref, b1_ref, w2_ref, b2_ref, out_ref,
             pooled_ref, cnt_ref):
        i = pl.program_id(0)

        @pl.when(i == 0)
        def _():
            pooled_ref[...] = jnp.zeros_like(pooled_ref)
            cnt_ref[...] = jnp.zeros_like(cnt_ref)

        h = jnp.concatenate([h_ref[0], h_ref[1]], axis=-1)
        bvec = bat_ref[0, 0, :]
        onehot = (bvec[:, None]
                  == lax.broadcasted_iota(jnp.int32, (R, G), 1)
                  ).astype(jnp.float32)
        pooled_ref[...] += lax.dot_general(
            onehot, h, (((0,), (0,)), ((), ())),
            preferred_element_type=jnp.float32)
        cnt_ref[0, :] += jnp.sum(onehot, axis=0)

        @pl.when(i == nblk - 1)
        def _():
            cnt = jnp.maximum(cnt_ref[0, :], 1.0)
            pooled = pooled_ref[...] / cnt[:, None]
            t = (jnp.dot(pooled, w1_ref[...],
                         preferred_element_type=jnp.float32) + b1_ref[...])
            out_ref[...] = (jnp.dot(t, w2_ref[...],
                                    preferred_element_type=jnp.float32)
                            + b2_ref[...])

    return pl.pallas_call(
        body,
        grid=(nblk,),
        in_specs=[
            pl.BlockSpec((2, R, 128), lambda i: (0, i, 0)),
            pl.BlockSpec((1, 1, R), lambda i: (i, 0, 0)),
            pl.BlockSpec((256, 128), lambda i: (0, 0)),
            pl.BlockSpec((1, 128), lambda i: (0, 0)),
            pl.BlockSpec((128, 128), lambda i: (0, 0)),
            pl.BlockSpec((1, 128), lambda i: (0, 0)),
        ],
        out_specs=pl.BlockSpec((G, 128), lambda i: (0, 0)),
        out_shape=jax.ShapeDtypeStruct((G, 128), jnp.float32),
        scratch_shapes=[
            pltpu.VMEM((G, 256), jnp.float32),
            pltpu.VMEM((8, 128), jnp.float32),
        ],
    )(h2, batch3, w1, b1.reshape(1, 128), w2p, b2p)


def kernel(x, edge_index, batch, Wrel1, Wroot1, b1, Wrel2, Wroot2, b2,
           Wrel3, Wroot3, b3, Wrel4, Wroot4, b4, lin1_W, lin1_b,
           lin2_W, lin2_b):
    src, dst = edge_index[0], edge_index[1]
    src4 = src.reshape(2, NSC, NCH_E // NB, NB, K)
    dst4 = dst.reshape(2, NSC, NCH_E // NB, NB, K)
    src4c = jnp.stack([src, src + N]).reshape(2, NSC, NCH_C // NB, NB, K)
    dst4c = jnp.stack([dst, dst]).reshape(2, NSC, NCH_C // NB, NB, K)
    z128 = jnp.zeros((NPS, 128), jnp.float32)
    batch3 = batch.reshape(N // R, 1, R)

    agg = _sc_agg_edges(x, src4, dst4, z128)
    h = _tc_layer(agg, x, Wrel1, Wroot1, b1,
                  relu=True, agg_sum=True, out_halves=False)
    agg = _sc_agg_edges(h, src4, dst4, z128)
    h = _tc_layer(agg, h, Wrel2, Wroot2, b2,
                  relu=True, agg_sum=True, out_halves=True)
    agg = _sc_agg_cols(h.reshape(2 * N, 128), src4c, dst4c, z128)
    h = _tc_layer(agg, h, Wrel3, Wroot3, b3,
                  relu=True, agg_sum=False, out_halves=True)
    agg = _sc_agg_cols(h.reshape(2 * N, 128), src4c, dst4c, z128)
    h = _tc_layer(agg, h, Wrel4, Wroot4, b4,
                  relu=False, agg_sum=False, out_halves=True)

    w2p = jnp.zeros((128, 128), jnp.float32).at[:, :10].set(lin2_W)
    b2p = jnp.zeros((1, 128), jnp.float32).at[0, :10].set(lin2_b)
    out = _tc_pool_mlp(h, batch3, lin1_W, lin1_b, w2p, b2p)
    return out[:, :10]


# speedup vs baseline: 9.6451x; 1.0291x over previous
import functools

import jax
import jax.numpy as jnp
from jax import lax
from jax.experimental import pallas as pl
from jax.experimental.pallas import tpu as pltpu
from jax.experimental.pallas import tpu_sc as plsc

N = 10000
E = 320000
G = 128
NSC = 16
K = 80
NPAD = 10240
NPS = NPAD // NSC
R = 1000
NCH_C = E // NSC // K
NCH_E = E // 2 // NSC // K
NB = 25

_MESH = plsc.VectorSubcoreMesh(core_axis_name="c", subcore_axis_name="s")


def _edge_loop(c, s, src5, dst5, hh, acc, srcv, dstv, rows, gsems, ssems,
               nblk):
    NQ = NB // 4

    def gissue(j, t):
        pltpu.async_copy(hh.at[srcv.at[j]], rows[t], gsems[t])

    def gwait(j, t):
        pltpu.make_async_copy(hh.at[srcv.at[j]], rows[t], gsems[t]).wait()

    def sissue(j, t):
        pltpu.async_copy(rows[t], acc.at[dstv.at[j]], ssems[t], add=True)

    def swait(j, t):
        pltpu.make_async_copy(rows[t], acc.at[dstv.at[j]], ssems[t]).wait()

    def blk_body(blk, carry):
        pltpu.sync_copy(src5.at[c, s, blk], srcv)
        pltpu.sync_copy(dst5.at[c, s, blk], dstv)
        gissue(0, 0)
        gissue(1, 1)

        def quad(q, carry2):
            j0 = 4 * q
            for t in range(4):
                j = j0 + t
                t2 = (t + 2) % 4
                gwait(j, t)
                sissue(j, t)
                if t >= 2:
                    swait(j - 2, t2)
                else:
                    @pl.when(q > 0)
                    def _():
                        swait(j - 2, t2)
                if t == 3:
                    @pl.when(q < NQ - 1)
                    def _():
                        gissue(j + 2, t2)
                else:
                    gissue(j + 2, t2)
            return carry2

        lax.fori_loop(0, NQ, quad, carry)
        jt = NB - 1
        gwait(jt, 0)
        sissue(jt, 0)
        swait(jt - 2, 2)
        swait(jt - 1, 3)
        swait(jt, 0)
        return carry

    lax.fori_loop(0, nblk, blk_body, 0)


def _make_sc_agg(nch):
    nblk = nch // NB

    @functools.partial(
        pl.kernel,
        out_type=jax.ShapeDtypeStruct((2, NPAD, 128), jnp.float32),
        mesh=_MESH,
        scratch_types=[
            pltpu.VMEM((NB, K), jnp.int32),
            pltpu.VMEM((NB, K), jnp.int32),
            [pltpu.VMEM((K, 128), jnp.float32)] * 4,
            [pltpu.SemaphoreType.DMA] * 4,
            [pltpu.SemaphoreType.DMA] * 4,
            pltpu.VMEM_SHARED((NPAD, 128), jnp.float32),
        ],
    )
    def agg(table, src5, dst5, zrows, out, srcv, dstv, rows, gsems, ssems,
            acc):
        c = lax.axis_index("c")
        s = lax.axis_index("s")
        pltpu.sync_copy(zrows, acc.at[pl.ds(s * NPS, NPS)])
        plsc.subcore_barrier()
        _edge_loop(c, s, src5, dst5, table, acc, srcv, dstv, rows,
                   gsems, ssems, nblk)
        plsc.subcore_barrier()
        pltpu.sync_copy(acc.at[pl.ds(s * NPS, NPS)],
                        out.at[c, pl.ds(s * NPS, NPS)])

    return agg


_sc_agg_edges = _make_sc_agg(NCH_E)
_sc_agg_cols = _make_sc_agg(NCH_C)


def _tc_layer(agg2, h, wrel, wroot, b, relu, agg_sum, out_halves):
    fin, fout = wrel.shape
    h_halves = h.ndim == 3
    nblk = N // R

    def body(agg_ref, h_ref, wrel_ref, wroot_ref, b_ref, out_ref):
        if agg_sum:
            agg = agg_ref[0] + agg_ref[1]
        else:
            agg = jnp.concatenate([agg_ref[0], agg_ref[1]], axis=-1)
        if h_halves:
            hb = jnp.concatenate([h_ref[0], h_ref[1]], axis=-1)
        else:
            hb = h_ref[...]
        y = (jnp.dot(agg, wrel_ref[...], preferred_element_type=jnp.float32)
             + jnp.dot(hb, wroot_ref[...], preferred_element_type=jnp.float32)
             + b_ref[...])
        if relu:
            y = jnp.maximum(y, 0.0)
        if out_halves:
            out_ref[0] = y[:, :fout // 2]
            out_ref[1] = y[:, fout // 2:]
        else:
            out_ref[...] = y

    h_spec = (pl.BlockSpec((2, R, 128), lambda i: (0, i, 0)) if h_halves
              else pl.BlockSpec((R, fin), lambda i: (i, 0)))
    out_spec = (pl.BlockSpec((2, R, fout // 2), lambda i: (0, i, 0))
                if out_halves else pl.BlockSpec((R, fout), lambda i: (i, 0)))
    out_shape = (jax.ShapeDtypeStruct((2, N, fout // 2), jnp.float32)
                 if out_halves else jax.ShapeDtypeStruct((N, fout), jnp.float32))
    return pl.pallas_call(
        body,
        grid=(nblk,),
        in_specs=[
            pl.BlockSpec((2, R, 128), lambda i: (0, i, 0)),
            h_spec,
            pl.BlockSpec((fin, fout), lambda i: (0, 0)),
            pl.BlockSpec((fin, fout), lambda i: (0, 0)),
            pl.BlockSpec((1, fout), lambda i: (0, 0)),
        ],
        out_specs=out_spec,
        out_shape=out_shape,
    )(agg2, h, wrel, wroot, b.reshape(1, fout))


def _tc_pool_mlp(h2, batch3, w1, b1, w2p, b2p):
    nblk = N // R

    def body(h_ref, bat_ref, w1_ref, b1_ref, w2_ref, b2_ref, out_ref,
             pooled_ref, cnt_ref):
        i = pl.program_id(0)

        @pl.when(i == 0)
        def _():
            pooled_ref[...] = jnp.zeros_like(pooled_ref)
            cnt_ref[...] = jnp.zeros_like(cnt_ref)

        h = jnp.concatenate([h_ref[0], h_ref[1]], axis=-1)
        bvec = bat_ref[0, 0, :]
        onehot = (bvec[:, None]
                  == lax.broadcasted_iota(jnp.int32, (R, G), 1)
                  ).astype(jnp.float32)
        pooled_ref[...] += lax.dot_general(
            onehot, h, (((0,), (0,)), ((), ())),
            preferred_element_type=jnp.float32)
        cnt_ref[0, :] += jnp.sum(onehot, axis=0)

        @pl.when(i == nblk - 1)
        def _():
            cnt = jnp.maximum(cnt_ref[0, :], 1.0)
            pooled = pooled_ref[...] / cnt[:, None]
            t = (jnp.dot(pooled, w1_ref[...],
                         preferred_element_type=jnp.float32) + b1_ref[...])
            out_ref[...] = (jnp.dot(t, w2_ref[...],
                                    preferred_element_type=jnp.float32)
                            + b2_ref[...])

    return pl.pallas_call(
        body,
        grid=(nblk,),
        in_specs=[
            pl.BlockSpec((2, R, 128), lambda i: (0, i, 0)),
            pl.BlockSpec((1, 1, R), lambda i: (i, 0, 0)),
            pl.BlockSpec((256, 128), lambda i: (0, 0)),
            pl.BlockSpec((1, 128), lambda i: (0, 0)),
            pl.BlockSpec((128, 128), lambda i: (0, 0)),
            pl.BlockSpec((1, 128), lambda i: (0, 0)),
        ],
        out_specs=pl.BlockSpec((G, 128), lambda i: (0, 0)),
        out_shape=jax.ShapeDtypeStruct((G, 128), jnp.float32),
        scratch_shapes=[
            pltpu.VMEM((G, 256), jnp.float32),
            pltpu.VMEM((8, 128), jnp.float32),
        ],
    )(h2, batch3, w1, b1.reshape(1, 128), w2p, b2p)


def kernel(x, edge_index, batch, Wrel1, Wroot1, b1, Wrel2, Wroot2, b2,
           Wrel3, Wroot3, b3, Wrel4, Wroot4, b4, lin1_W, lin1_b,
           lin2_W, lin2_b):
    src, dst = edge_index[0], edge_index[1]
    src4 = src.reshape(2, NSC, NCH_E // NB, NB, K)
    dst4 = dst.reshape(2, NSC, NCH_E // NB, NB, K)
    src4c = jnp.stack([src, src + N]).reshape(2, NSC, NCH_C // NB, NB, K)
    dst4c = jnp.stack([dst, dst]).reshape(2, NSC, NCH_C // NB, NB, K)
    z128 = jnp.zeros((NPS, 128), jnp.float32)
    batch3 = batch.reshape(N // R, 1, R)

    agg = _sc_agg_edges(x, src4, dst4, z128)
    h = _tc_layer(agg, x, Wrel1, Wroot1, b1,
                  relu=True, agg_sum=True, out_halves=False)
    agg = _sc_agg_edges(h, src4, dst4, z128)
    h = _tc_layer(agg, h, Wrel2, Wroot2, b2,
                  relu=True, agg_sum=True, out_halves=True)
    agg = _sc_agg_cols(h.reshape(2 * N, 128), src4c, dst4c, z128)
    h = _tc_layer(agg, h, Wrel3, Wroot3, b3,
                  relu=True, agg_sum=False, out_halves=True)
    agg = _sc_agg_cols(h.reshape(2 * N, 128), src4c, dst4c, z128)
    h = _tc_layer(agg, h, Wrel4, Wroot4, b4,
                  relu=False, agg_sum=False, out_halves=True)

    w2p = jnp.zeros((128, 128), jnp.float32).at[:, :10].set(lin2_W)
    b2p = jnp.zeros((1, 128), jnp.float32).at[0, :10].set(lin2_b)
    out = _tc_pool_mlp(h, batch3, lin1_W, lin1_b, w2p, b2p)
    return out[:, :10]


# speedup vs baseline: 9.8447x; 1.0207x over previous
import functools

import jax
import jax.numpy as jnp
from jax import lax
from jax.experimental import pallas as pl
from jax.experimental.pallas import tpu as pltpu
from jax.experimental.pallas import tpu_sc as plsc

N = 10000
E = 320000
G = 128
NSC = 16
K = 80
NPAD = 10240
NPS = NPAD // NSC
R = 1000
NCH_C = E // NSC // K
NCH_E = E // 2 // NSC // K
NB = 25

_MESH = plsc.VectorSubcoreMesh(core_axis_name="c", subcore_axis_name="s")


def _edge_loop(c, s, src5, dst5, hh, acc, srcv, dstv, rows, gsems, ssems,
               nblk):
    NQ = NB // 4

    def gissue(j, t):
        pltpu.async_copy(hh.at[srcv.at[j]], rows[t], gsems[t])

    def gwait(j, t):
        pltpu.make_async_copy(hh.at[srcv.at[j]], rows[t], gsems[t]).wait()

    def sissue(j, t):
        pltpu.async_copy(rows[t], acc.at[dstv.at[j]], ssems[t], add=True)

    def swait(j, t):
        pltpu.make_async_copy(rows[t], acc.at[dstv.at[j]], ssems[t]).wait()

    def blk_body(blk, carry):
        pltpu.sync_copy(src5.at[c, s, blk], srcv)
        pltpu.sync_copy(dst5.at[c, s, blk], dstv)
        gissue(0, 0)
        gissue(1, 1)

        def quad(q, carry2):
            j0 = 4 * q
            for t in range(4):
                j = j0 + t
                t2 = (t + 2) % 4
                gwait(j, t)
                sissue(j, t)
                if t >= 2:
                    swait(j - 2, t2)
                else:
                    @pl.when(q > 0)
                    def _():
                        swait(j - 2, t2)
                if t == 3:
                    @pl.when(q < NQ - 1)
                    def _():
                        gissue(j + 2, t2)
                else:
                    gissue(j + 2, t2)
            return carry2

        lax.fori_loop(0, NQ, quad, carry)
        jt = NB - 1
        gwait(jt, 0)
        sissue(jt, 0)
        swait(jt - 2, 2)
        swait(jt - 1, 3)
        swait(jt, 0)
        return carry

    lax.fori_loop(0, nblk, blk_body, 0)


def _make_sc_agg(nch):
    nblk = nch // NB

    @functools.partial(
        pl.kernel,
        out_type=jax.ShapeDtypeStruct((2, NPAD, 128), jnp.float32),
        mesh=_MESH,
        scratch_types=[
            pltpu.VMEM((NB, K), jnp.int32),
            pltpu.VMEM((NB, K), jnp.int32),
            [pltpu.VMEM((K, 128), jnp.float32)] * 4,
            [pltpu.SemaphoreType.DMA] * 4,
            [pltpu.SemaphoreType.DMA] * 4,
            pltpu.VMEM_SHARED((NPAD, 128), jnp.float32),
        ],
    )
    def agg(table, src5, dst5, out, srcv, dstv, rows, gsems, ssems, acc):
        c = lax.axis_index("c")
        s = lax.axis_index("s")
        zv = jnp.zeros((16,), jnp.float32)

        def zrow(i, carry):
            for l in range(8):
                rows[0][i, pl.ds(l * 16, 16)] = zv
            return carry

        lax.fori_loop(0, K, zrow, 0)
        for r in range(NPS // K):
            pltpu.sync_copy(rows[0], acc.at[pl.ds(s * NPS + r * K, K)])
        plsc.subcore_barrier()
        _edge_loop(c, s, src5, dst5, table, acc, srcv, dstv, rows,
                   gsems, ssems, nblk)
        plsc.subcore_barrier()
        pltpu.sync_copy(acc.at[pl.ds(s * NPS, NPS)],
                        out.at[c, pl.ds(s * NPS, NPS)])

    return agg


_sc_agg_edges = _make_sc_agg(NCH_E)
_sc_agg_cols = _make_sc_agg(NCH_C)


def _tc_layer(agg2, h, wrel, wroot, b, relu, agg_sum, out_halves):
    fin, fout = wrel.shape
    h_halves = h.ndim == 3
    nblk = N // R

    def body(agg_ref, h_ref, wrel_ref, wroot_ref, b_ref, out_ref):
        if agg_sum:
            agg = agg_ref[0] + agg_ref[1]
        else:
            agg = jnp.concatenate([agg_ref[0], agg_ref[1]], axis=-1)
        if h_halves:
            hb = jnp.concatenate([h_ref[0], h_ref[1]], axis=-1)
        else:
            hb = h_ref[...]
        y = (jnp.dot(agg, wrel_ref[...], preferred_element_type=jnp.float32)
             + jnp.dot(hb, wroot_ref[...], preferred_element_type=jnp.float32)
             + b_ref[...])
        if relu:
            y = jnp.maximum(y, 0.0)
        if out_halves:
            out_ref[0] = y[:, :fout // 2]
            out_ref[1] = y[:, fout // 2:]
        else:
            out_ref[...] = y

    h_spec = (pl.BlockSpec((2, R, 128), lambda i: (0, i, 0)) if h_halves
              else pl.BlockSpec((R, fin), lambda i: (i, 0)))
    out_spec = (pl.BlockSpec((2, R, fout // 2), lambda i: (0, i, 0))
                if out_halves else pl.BlockSpec((R, fout), lambda i: (i, 0)))
    out_shape = (jax.ShapeDtypeStruct((2, N, fout // 2), jnp.float32)
                 if out_halves else jax.ShapeDtypeStruct((N, fout), jnp.float32))
    return pl.pallas_call(
        body,
        grid=(nblk,),
        in_specs=[
            pl.BlockSpec((2, R, 128), lambda i: (0, i, 0)),
            h_spec,
            pl.BlockSpec((fin, fout), lambda i: (0, 0)),
            pl.BlockSpec((fin, fout), lambda i: (0, 0)),
            pl.BlockSpec((1, fout), lambda i: (0, 0)),
        ],
        out_specs=out_spec,
        out_shape=out_shape,
    )(agg2, h, wrel, wroot, b.reshape(1, fout))


def _tc_pool_mlp(h2, batch3, w1, b1, w2p, b2p):
    nblk = N // R

    def body(h_ref, bat_ref, w1_ref, b1_ref, w2_ref, b2_ref, out_ref,
             pooled_ref, cnt_ref):
        i = pl.program_id(0)

        @pl.when(i == 0)
        def _():
            pooled_ref[...] = jnp.zeros_like(pooled_ref)
            cnt_ref[...] = jnp.zeros_like(cnt_ref)

        h = jnp.concatenate([h_ref[0], h_ref[1]], axis=-1)
        bvec = bat_ref[0, 0, :]
        onehot = (bvec[:, None]
                  == lax.broadcasted_iota(jnp.int32, (R, G), 1)
                  ).astype(jnp.float32)
        pooled_ref[...] += lax.dot_general(
            onehot, h, (((0,), (0,)), ((), ())),
            preferred_element_type=jnp.float32)
        cnt_ref[0, :] += jnp.sum(onehot, axis=0)

        @pl.when(i == nblk - 1)
        def _():
            cnt = jnp.maximum(cnt_ref[0, :], 1.0)
            pooled = pooled_ref[...] / cnt[:, None]
            t = (jnp.dot(pooled, w1_ref[...],
                         preferred_element_type=jnp.float32) + b1_ref[...])
            out_ref[...] = (jnp.dot(t, w2_ref[...],
                                    preferred_element_type=jnp.float32)
                            + b2_ref[...])

    return pl.pallas_call(
        body,
        grid=(nblk,),
        in_specs=[
            pl.BlockSpec((2, R, 128), lambda i: (0, i, 0)),
            pl.BlockSpec((1, 1, R), lambda i: (i, 0, 0)),
            pl.BlockSpec((256, 128), lambda i: (0, 0)),
            pl.BlockSpec((1, 128), lambda i: (0, 0)),
            pl.BlockSpec((128, 128), lambda i: (0, 0)),
            pl.BlockSpec((1, 128), lambda i: (0, 0)),
        ],
        out_specs=pl.BlockSpec((G, 128), lambda i: (0, 0)),
        out_shape=jax.ShapeDtypeStruct((G, 128), jnp.float32),
        scratch_shapes=[
            pltpu.VMEM((G, 256), jnp.float32),
            pltpu.VMEM((8, 128), jnp.float32),
        ],
    )(h2, batch3, w1, b1.reshape(1, 128), w2p, b2p)


def kernel(x, edge_index, batch, Wrel1, Wroot1, b1, Wrel2, Wroot2, b2,
           Wrel3, Wroot3, b3, Wrel4, Wroot4, b4, lin1_W, lin1_b,
           lin2_W, lin2_b):
    src, dst = edge_index[0], edge_index[1]
    src4 = src.reshape(2, NSC, NCH_E // NB, NB, K)
    dst4 = dst.reshape(2, NSC, NCH_E // NB, NB, K)
    src4c = jnp.stack([src, src + N]).reshape(2, NSC, NCH_C // NB, NB, K)
    dst4c = jnp.stack([dst, dst]).reshape(2, NSC, NCH_C // NB, NB, K)
    batch3 = batch.reshape(N // R, 1, R)

    agg = _sc_agg_edges(x, src4, dst4)
    h = _tc_layer(agg, x, Wrel1, Wroot1, b1,
                  relu=True, agg_sum=True, out_halves=False)
    agg = _sc_agg_edges(h, src4, dst4)
    h = _tc_layer(agg, h, Wrel2, Wroot2, b2,
                  relu=True, agg_sum=True, out_halves=True)
    agg = _sc_agg_cols(h.reshape(2 * N, 128), src4c, dst4c)
    h = _tc_layer(agg, h, Wrel3, Wroot3, b3,
                  relu=True, agg_sum=False, out_halves=True)
    agg = _sc_agg_cols(h.reshape(2 * N, 128), src4c, dst4c)
    h = _tc_layer(agg, h, Wrel4, Wroot4, b4,
                  relu=False, agg_sum=False, out_halves=True)

    w2p = jnp.zeros((128, 128), jnp.float32).at[:, :10].set(lin2_W)
    b2p = jnp.zeros((1, 128), jnp.float32).at[0, :10].set(lin2_b)
    out = _tc_pool_mlp(h, batch3, lin1_W, lin1_b, w2p, b2p)
    return out[:, :10]


# speedup vs baseline: 10.6739x; 1.0842x over previous
import functools

import jax
import jax.numpy as jnp
from jax import lax
from jax.experimental import pallas as pl
from jax.experimental.pallas import tpu as pltpu
from jax.experimental.pallas import tpu_sc as plsc

N = 10000
E = 320000
G = 128
NSC = 16
K = 80
NPAD = 10240
NPS = NPAD // NSC
R = 1000
NCH_C = E // NSC // K
NCH_E = E // 2 // NSC // K
NB = 25

_MESH = plsc.VectorSubcoreMesh(core_axis_name="c", subcore_axis_name="s")


def _edge_loop(c, s, src5, dst5, hh, acc, srcv, dstv, rows, gsems, ssems,
               nblk):
    NQ = NB // 4

    def gissue(j, t):
        pltpu.async_copy(hh.at[srcv.at[j]], rows[t], gsems[t])

    def gwait(j, t):
        pltpu.make_async_copy(hh.at[srcv.at[j]], rows[t], gsems[t]).wait()

    def sissue(j, t):
        pltpu.async_copy(rows[t], acc.at[dstv.at[j]], ssems[t], add=True)

    def swait(j, t):
        pltpu.make_async_copy(rows[t], acc.at[dstv.at[j]], ssems[t]).wait()

    def blk_body(blk, carry):
        pltpu.sync_copy(src5.at[c, s, blk], srcv)
        pltpu.sync_copy(dst5.at[c, s, blk], dstv)
        gissue(0, 0)
        gissue(1, 1)

        gissue(2, 2)

        def quad(q, carry2):
            j0 = 4 * q
            for t in range(4):
                j = j0 + t
                t3 = (t + 3) % 4
                gwait(j, t)
                sissue(j, t)
                if t >= 1:
                    swait(j - 1, t3)
                else:
                    @pl.when(q > 0)
                    def _():
                        swait(j - 1, t3)
                if t >= 2:
                    @pl.when(q < NQ - 1)
                    def _():
                        gissue(j + 3, t3)
                else:
                    gissue(j + 3, t3)
            return carry2

        lax.fori_loop(0, NQ, quad, carry)
        jt = NB - 1
        gwait(jt, 0)
        sissue(jt, 0)
        swait(jt - 1, 3)
        swait(jt, 0)
        return carry

    lax.fori_loop(0, nblk, blk_body, 0)


def _make_sc_agg(nch):
    nblk = nch // NB

    @functools.partial(
        pl.kernel,
        out_type=jax.ShapeDtypeStruct((2, NPAD, 128), jnp.float32),
        mesh=_MESH,
        scratch_types=[
            pltpu.VMEM((NB, K), jnp.int32),
            pltpu.VMEM((NB, K), jnp.int32),
            [pltpu.VMEM((K, 128), jnp.float32)] * 4,
            [pltpu.SemaphoreType.DMA] * 4,
            [pltpu.SemaphoreType.DMA] * 4,
            pltpu.VMEM_SHARED((NPAD, 128), jnp.float32),
        ],
    )
    def agg(table, src5, dst5, out, srcv, dstv, rows, gsems, ssems, acc):
        c = lax.axis_index("c")
        s = lax.axis_index("s")
        zv = jnp.zeros((16,), jnp.float32)

        def zrow(i, carry):
            for l in range(8):
                rows[0][i, pl.ds(l * 16, 16)] = zv
            return carry

        lax.fori_loop(0, K, zrow, 0)
        for r in range(NPS // K):
            pltpu.sync_copy(rows[0], acc.at[pl.ds(s * NPS + r * K, K)])
        plsc.subcore_barrier()
        _edge_loop(c, s, src5, dst5, table, acc, srcv, dstv, rows,
                   gsems, ssems, nblk)
        plsc.subcore_barrier()
        pltpu.sync_copy(acc.at[pl.ds(s * NPS, NPS)],
                        out.at[c, pl.ds(s * NPS, NPS)])

    return agg


_sc_agg_edges = _make_sc_agg(NCH_E)
_sc_agg_cols = _make_sc_agg(NCH_C)


def _tc_layer(agg2, h, wrel, wroot, b, relu, agg_sum, out_halves):
    fin, fout = wrel.shape
    h_halves = h.ndim == 3
    nblk = N // R

    def body(agg_ref, h_ref, wrel_ref, wroot_ref, b_ref, out_ref):
        if agg_sum:
            agg = agg_ref[0] + agg_ref[1]
        else:
            agg = jnp.concatenate([agg_ref[0], agg_ref[1]], axis=-1)
        if h_halves:
            hb = jnp.concatenate([h_ref[0], h_ref[1]], axis=-1)
        else:
            hb = h_ref[...]
        y = (jnp.dot(agg, wrel_ref[...], preferred_element_type=jnp.float32)
             + jnp.dot(hb, wroot_ref[...], preferred_element_type=jnp.float32)
             + b_ref[...])
        if relu:
            y = jnp.maximum(y, 0.0)
        if out_halves:
            out_ref[0] = y[:, :fout // 2]
            out_ref[1] = y[:, fout // 2:]
        else:
            out_ref[...] = y

    h_spec = (pl.BlockSpec((2, R, 128), lambda i: (0, i, 0)) if h_halves
              else pl.BlockSpec((R, fin), lambda i: (i, 0)))
    out_spec = (pl.BlockSpec((2, R, fout // 2), lambda i: (0, i, 0))
                if out_halves else pl.BlockSpec((R, fout), lambda i: (i, 0)))
    out_shape = (jax.ShapeDtypeStruct((2, N, fout // 2), jnp.float32)
                 if out_halves else jax.ShapeDtypeStruct((N, fout), jnp.float32))
    return pl.pallas_call(
        body,
        grid=(nblk,),
        in_specs=[
            pl.BlockSpec((2, R, 128), lambda i: (0, i, 0)),
            h_spec,
            pl.BlockSpec((fin, fout), lambda i: (0, 0)),
            pl.BlockSpec((fin, fout), lambda i: (0, 0)),
            pl.BlockSpec((1, fout), lambda i: (0, 0)),
        ],
        out_specs=out_spec,
        out_shape=out_shape,
    )(agg2, h, wrel, wroot, b.reshape(1, fout))


def _tc_pool_mlp(h2, batch3, w1, b1, w2p, b2p):
    nblk = N // R

    def body(h_ref, bat_ref, w1_ref, b1_ref, w2_ref, b2_ref, out_ref,
             pooled_ref, cnt_ref):
        i = pl.program_id(0)

        @pl.when(i == 0)
        def _():
            pooled_ref[...] = jnp.zeros_like(pooled_ref)
            cnt_ref[...] = jnp.zeros_like(cnt_ref)

        h = jnp.concatenate([h_ref[0], h_ref[1]], axis=-1)
        bvec = bat_ref[0, 0, :]
        onehot = (bvec[:, None]
                  == lax.broadcasted_iota(jnp.int32, (R, G), 1)
                  ).astype(jnp.float32)
        pooled_ref[...] += lax.dot_general(
            onehot, h, (((0,), (0,)), ((), ())),
            preferred_element_type=jnp.float32)
        cnt_ref[0, :] += jnp.sum(onehot, axis=0)

        @pl.when(i == nblk - 1)
        def _():
            cnt = jnp.maximum(cnt_ref[0, :], 1.0)
            pooled = pooled_ref[...] / cnt[:, None]
            t = (jnp.dot(pooled, w1_ref[...],
                         preferred_element_type=jnp.float32) + b1_ref[...])
            out_ref[...] = (jnp.dot(t, w2_ref[...],
                                    preferred_element_type=jnp.float32)
                            + b2_ref[...])

    return pl.pallas_call(
        body,
        grid=(nblk,),
        in_specs=[
            pl.BlockSpec((2, R, 128), lambda i: (0, i, 0)),
            pl.BlockSpec((1, 1, R), lambda i: (i, 0, 0)),
            pl.BlockSpec((256, 128), lambda i: (0, 0)),
            pl.BlockSpec((1, 128), lambda i: (0, 0)),
            pl.BlockSpec((128, 128), lambda i: (0, 0)),
            pl.BlockSpec((1, 128), lambda i: (0, 0)),
        ],
        out_specs=pl.BlockSpec((G, 128), lambda i: (0, 0)),
        out_shape=jax.ShapeDtypeStruct((G, 128), jnp.float32),
        scratch_shapes=[
            pltpu.VMEM((G, 256), jnp.float32),
            pltpu.VMEM((8, 128), jnp.float32),
        ],
    )(h2, batch3, w1, b1.reshape(1, 128), w2p, b2p)


def kernel(x, edge_index, batch, Wrel1, Wroot1, b1, Wrel2, Wroot2, b2,
           Wrel3, Wroot3, b3, Wrel4, Wroot4, b4, lin1_W, lin1_b,
           lin2_W, lin2_b):
    src, dst = edge_index[0], edge_index[1]
    src4 = src.reshape(2, NSC, NCH_E // NB, NB, K)
    dst4 = dst.reshape(2, NSC, NCH_E // NB, NB, K)
    src4c = jnp.stack([src, src + N]).reshape(2, NSC, NCH_C // NB, NB, K)
    dst4c = jnp.stack([dst, dst]).reshape(2, NSC, NCH_C // NB, NB, K)
    batch3 = batch.reshape(N // R, 1, R)

    agg = _sc_agg_edges(x, src4, dst4)
    h = _tc_layer(agg, x, Wrel1, Wroot1, b1,
                  relu=True, agg_sum=True, out_halves=False)
    agg = _sc_agg_edges(h, src4, dst4)
    h = _tc_layer(agg, h, Wrel2, Wroot2, b2,
                  relu=True, agg_sum=True, out_halves=True)
    agg = _sc_agg_cols(h.reshape(2 * N, 128), src4c, dst4c)
    h = _tc_layer(agg, h, Wrel3, Wroot3, b3,
                  relu=True, agg_sum=False, out_halves=True)
    agg = _sc_agg_cols(h.reshape(2 * N, 128), src4c, dst4c)
    h = _tc_layer(agg, h, Wrel4, Wroot4, b4,
                  relu=False, agg_sum=False, out_halves=True)

    w2p = jnp.zeros((128, 128), jnp.float32).at[:, :10].set(lin2_W)
    b2p = jnp.zeros((1, 128), jnp.float32).at[0, :10].set(lin2_b)
    out = _tc_pool_mlp(h, batch3, lin1_W, lin1_b, w2p, b2p)
    return out[:, :10]


# speedup vs baseline: 11.0480x; 1.0351x over previous
import functools

import jax
import jax.numpy as jnp
from jax import lax
from jax.experimental import pallas as pl
from jax.experimental.pallas import tpu as pltpu
from jax.experimental.pallas import tpu_sc as plsc

N = 10000
E = 320000
G = 128
NSC = 16
K = 80
NPAD = 10240
NPS = NPAD // NSC
R = 1000
NCH_C = E // NSC // K
NCH_E = E // 2 // NSC // K
NB = 25

_MESH = plsc.VectorSubcoreMesh(core_axis_name="c", subcore_axis_name="s")


def _edge_loop(c, s, sd, hh, acc, sdv, rows, gsems, ssems, nblk):
    NQ = NB // 4
    srcv = sdv.at[0]
    dstv = sdv.at[1]

    def gissue(j, t):
        pltpu.async_copy(hh.at[srcv.at[j]], rows[t], gsems[t])

    def gwait(j, t):
        pltpu.make_async_copy(hh.at[srcv.at[j]], rows[t], gsems[t]).wait()

    def sissue(j, t):
        pltpu.async_copy(rows[t], acc.at[dstv.at[j]], ssems[t], add=True)

    def swait(j, t):
        pltpu.make_async_copy(rows[t], acc.at[dstv.at[j]], ssems[t]).wait()

    def blk_body(blk, carry):
        pltpu.sync_copy(sd.at[c, s, blk], sdv)
        gissue(0, 0)
        gissue(1, 1)
        gissue(2, 2)

        def quad(q, carry2):
            j0 = 4 * q
            for t in range(4):
                j = j0 + t
                t3 = (t + 3) % 4
                gwait(j, t)
                sissue(j, t)
                if t >= 1:
                    swait(j - 1, t3)
                else:
                    @pl.when(q > 0)
                    def _():
                        swait(j - 1, t3)
                if t >= 2:
                    @pl.when(q < NQ - 1)
                    def _():
                        gissue(j + 3, t3)
                else:
                    gissue(j + 3, t3)
            return carry2

        lax.fori_loop(0, NQ, quad, 0)
        jt = NB - 1
        gwait(jt, 0)
        sissue(jt, 0)
        swait(jt - 1, 3)
        swait(jt, 0)
        return carry

    lax.fori_loop(0, nblk, blk_body, 0)


def _make_sc_agg(nch):
    nblk = nch // NB

    @functools.partial(
        pl.kernel,
        out_type=jax.ShapeDtypeStruct((2, NPAD, 128), jnp.float32),
        mesh=_MESH,
        scratch_types=[
            pltpu.VMEM((2, NB, K), jnp.int32),
            [pltpu.VMEM((K, 128), jnp.float32)] * 4,
            [pltpu.SemaphoreType.DMA] * 4,
            [pltpu.SemaphoreType.DMA] * 4,
            pltpu.VMEM_SHARED((NPAD, 128), jnp.float32),
        ],
    )
    def agg(table, sd, out, sdv, rows, gsems, ssems, acc):
        c = lax.axis_index("c")
        s = lax.axis_index("s")
        zv = jnp.zeros((16,), jnp.float32)

        def zrow(i, carry):
            for l in range(8):
                rows[0][i, pl.ds(l * 16, 16)] = zv
            return carry

        lax.fori_loop(0, K, zrow, 0)
        for r in range(NPS // K):
            pltpu.sync_copy(rows[0], acc.at[pl.ds(s * NPS + r * K, K)])
        plsc.subcore_barrier()
        _edge_loop(c, s, sd, table, acc, sdv, rows, gsems, ssems, nblk)
        plsc.subcore_barrier()
        pltpu.sync_copy(acc.at[pl.ds(s * NPS, NPS)],
                        out.at[c, pl.ds(s * NPS, NPS)])

    return agg


_sc_agg_edges = _make_sc_agg(NCH_E)
_sc_agg_cols = _make_sc_agg(NCH_C)


def _tc_layer(agg2, h, wrel, wroot, b, relu, agg_sum, out_halves):
    fin, fout = wrel.shape
    h_halves = h.ndim == 3
    nblk = N // R

    def body(agg_ref, h_ref, wrel_ref, wroot_ref, b_ref, out_ref):
        if agg_sum:
            agg = agg_ref[0] + agg_ref[1]
        else:
            agg = jnp.concatenate([agg_ref[0], agg_ref[1]], axis=-1)
        if h_halves:
            hb = jnp.concatenate([h_ref[0], h_ref[1]], axis=-1)
        else:
            hb = h_ref[...]
        y = (jnp.dot(agg, wrel_ref[...], preferred_element_type=jnp.float32)
             + jnp.dot(hb, wroot_ref[...], preferred_element_type=jnp.float32)
             + b_ref[...])
        if relu:
            y = jnp.maximum(y, 0.0)
        if out_halves:
            out_ref[0] = y[:, :fout // 2]
            out_ref[1] = y[:, fout // 2:]
        else:
            out_ref[...] = y

    h_spec = (pl.BlockSpec((2, R, 128), lambda i: (0, i, 0)) if h_halves
              else pl.BlockSpec((R, fin), lambda i: (i, 0)))
    out_spec = (pl.BlockSpec((2, R, fout // 2), lambda i: (0, i, 0))
                if out_halves else pl.BlockSpec((R, fout), lambda i: (i, 0)))
    out_shape = (jax.ShapeDtypeStruct((2, N, fout // 2), jnp.float32)
                 if out_halves else jax.ShapeDtypeStruct((N, fout), jnp.float32))
    return pl.pallas_call(
        body,
        grid=(nblk,),
        in_specs=[
            pl.BlockSpec((2, R, 128), lambda i: (0, i, 0)),
            h_spec,
            pl.BlockSpec((fin, fout), lambda i: (0, 0)),
            pl.BlockSpec((fin, fout), lambda i: (0, 0)),
            pl.BlockSpec((1, fout), lambda i: (0, 0)),
        ],
        out_specs=out_spec,
        out_shape=out_shape,
    )(agg2, h, wrel, wroot, b.reshape(1, fout))


def _tc_layer4_pool(agg2, h2, wrel, wroot, b, batch3, w1, b1, w2p, b2p):
    nblk = N // R

    def body(agg_ref, h_ref, wrel_ref, wroot_ref, b_ref, bat_ref, w1_ref,
             b1_ref, w2_ref, b2_ref, out_ref, pooled_ref, cnt_ref):
        i = pl.program_id(0)

        @pl.when(i == 0)
        def _():
            pooled_ref[...] = jnp.zeros_like(pooled_ref)
            cnt_ref[...] = jnp.zeros_like(cnt_ref)

        agg = jnp.concatenate([agg_ref[0], agg_ref[1]], axis=-1)
        hb = jnp.concatenate([h_ref[0], h_ref[1]], axis=-1)
        y = (jnp.dot(agg, wrel_ref[...], preferred_element_type=jnp.float32)
             + jnp.dot(hb, wroot_ref[...], preferred_element_type=jnp.float32)
             + b_ref[...])
        bvec = bat_ref[0, 0, :]
        onehot = (bvec[:, None]
                  == lax.broadcasted_iota(jnp.int32, (R, G), 1)
                  ).astype(jnp.float32)
        pooled_ref[...] += lax.dot_general(
            onehot, y, (((0,), (0,)), ((), ())),
            preferred_element_type=jnp.float32)
        cnt_ref[0, :] += jnp.sum(onehot, axis=0)

        @pl.when(i == nblk - 1)
        def _():
            cnt = jnp.maximum(cnt_ref[0, :], 1.0)
            pooled = pooled_ref[...] / cnt[:, None]
            t = (jnp.dot(pooled, w1_ref[...],
                         preferred_element_type=jnp.float32) + b1_ref[...])
            out_ref[...] = (jnp.dot(t, w2_ref[...],
                                    preferred_element_type=jnp.float32)
                            + b2_ref[...])

    return pl.pallas_call(
        body,
        grid=(nblk,),
        in_specs=[
            pl.BlockSpec((2, R, 128), lambda i: (0, i, 0)),
            pl.BlockSpec((2, R, 128), lambda i: (0, i, 0)),
            pl.BlockSpec((256, 256), lambda i: (0, 0)),
            pl.BlockSpec((256, 256), lambda i: (0, 0)),
            pl.BlockSpec((1, 256), lambda i: (0, 0)),
            pl.BlockSpec((1, 1, R), lambda i: (i, 0, 0)),
            pl.BlockSpec((256, 128), lambda i: (0, 0)),
            pl.BlockSpec((1, 128), lambda i: (0, 0)),
            pl.BlockSpec((128, 128), lambda i: (0, 0)),
            pl.BlockSpec((1, 128), lambda i: (0, 0)),
        ],
        out_specs=pl.BlockSpec((G, 128), lambda i: (0, 0)),
        out_shape=jax.ShapeDtypeStruct((G, 128), jnp.float32),
        scratch_shapes=[
            pltpu.VMEM((G, 256), jnp.float32),
            pltpu.VMEM((8, 128), jnp.float32),
        ],
    )(agg2, h2, wrel, wroot, b.reshape(1, 256), batch3,
      w1, b1.reshape(1, 128), w2p, b2p)


def kernel(x, edge_index, batch, Wrel1, Wroot1, b1, Wrel2, Wroot2, b2,
           Wrel3, Wroot3, b3, Wrel4, Wroot4, b4, lin1_W, lin1_b,
           lin2_W, lin2_b):
    src, dst = edge_index[0], edge_index[1]
    nbe = NCH_E // NB
    nbc = NCH_C // NB
    sd_e = jnp.stack([src.reshape(2, NSC, nbe, NB, K),
                      dst.reshape(2, NSC, nbe, NB, K)], axis=3)
    sd_c = jnp.stack([jnp.stack([src, src + N]).reshape(2, NSC, nbc, NB, K),
                      jnp.stack([dst, dst]).reshape(2, NSC, nbc, NB, K)],
                     axis=3)
    batch3 = batch.reshape(N // R, 1, R)

    agg = _sc_agg_edges(x, sd_e)
    h = _tc_layer(agg, x, Wrel1, Wroot1, b1,
                  relu=True, agg_sum=True, out_halves=False)
    agg = _sc_agg_edges(h, sd_e)
    h = _tc_layer(agg, h, Wrel2, Wroot2, b2,
                  relu=True, agg_sum=True, out_halves=True)
    agg = _sc_agg_cols(h.reshape(2 * N, 128), sd_c)
    h = _tc_layer(agg, h, Wrel3, Wroot3, b3,
                  relu=True, agg_sum=False, out_halves=True)
    agg = _sc_agg_cols(h.reshape(2 * N, 128), sd_c)

    w2p = jnp.zeros((128, 128), jnp.float32).at[:, :10].set(lin2_W)
    b2p = jnp.zeros((1, 128), jnp.float32).at[0, :10].set(lin2_b)
    out = _tc_layer4_pool(agg, h, Wrel4, Wroot4, b4, batch3,
                          lin1_W, lin1_b, w2p, b2p)
    return out[:, :10]
